# Initial kernel scaffold; baseline (speedup 1.0000x reference)
import jax
import jax.numpy as jnp
from jax.experimental import pallas as pl


def kernel(x, edge_index, W1, b1, W2, b2, Wc, bc):
    raise NotImplementedError("write your pallas kernel here")


# speedup vs baseline: 27.5568x; 27.5568x over previous
import functools

import jax
import jax.numpy as jnp
from jax import lax
from jax.experimental import pallas as pl
from jax.experimental.pallas import tpu as pltpu
from jax.experimental.pallas import tpu_sc as plsc

NC = 2
NS = 16
NW = NC * NS
LANES = 16

ROW = 125
DW = 16


def _mesh():
    return plsc.VectorSubcoreMesh(core_axis_name="c", subcore_axis_name="s")


def _make_deg_kernel(NP, E):
    n_rows = E // ROW
    BR = 8
    n_blocks = n_rows // BR
    T = n_blocks // NW
    RPT = NP // NS
    assert E % ROW == 0 and n_rows % BR == 0 and n_blocks % NW == 0
    assert RPT % 8 == 0

    @functools.partial(
        pl.kernel,
        out_type=jax.ShapeDtypeStruct((NC, NP, DW), jnp.float32),
        mesh=_mesh(),
        compiler_params=pltpu.CompilerParams(use_tc_tiling_on_sc=False),
        scratch_types=[
            pltpu.VMEM((BR, ROW), jnp.int32),
            pltpu.VMEM((ROW, DW), jnp.float32),
            pltpu.VMEM_SHARED((NP, DW), jnp.float32),
        ],
    )
    def deg_kernel(dst2d_hbm, ones_hbm, zeros1_hbm, out_hbm,
                   didxb, ones_v, acc):
        cid = lax.axis_index("c")
        sid = lax.axis_index("s")
        wid = sid * NC + cid

        row0 = sid * RPT
        pltpu.sync_copy(zeros1_hbm.at[pl.ds(row0, RPT)],
                        acc.at[pl.ds(row0, RPT)])
        pltpu.sync_copy(ones_hbm, ones_v)
        plsc.subcore_barrier()

        def blk_body(t, _):
            blk = wid + NW * t
            pltpu.sync_copy(dst2d_hbm.at[pl.ds(blk * BR, BR)], didxb)
            for j in range(BR):
                pltpu.sync_copy(ones_v, acc.at[didxb.at[j]], add=True)
            return 0

        lax.fori_loop(0, T, blk_body, 0)
        plsc.subcore_barrier()
        pltpu.sync_copy(acc.at[pl.ds(row0, RPT)],
                        out_hbm.at[cid, pl.ds(row0, RPT)])

    return deg_kernel


def _make_scatter_kernel(NP, E, H):
    n_rows = E // ROW
    BR = 8
    n_blocks = n_rows // BR
    T = n_blocks // NW
    RPT = NP // NS
    assert E % ROW == 0 and n_rows % BR == 0 and n_blocks % NW == 0
    assert RPT % 8 == 0

    @functools.partial(
        pl.kernel,
        out_type=jax.ShapeDtypeStruct((NC, NP, H), jnp.float32),
        mesh=_mesh(),
        scratch_types=[
            pltpu.VMEM((BR, ROW), jnp.int32),
            pltpu.VMEM((BR, ROW), jnp.int32),
            pltpu.VMEM((2, ROW, H), jnp.float32),
            pltpu.VMEM_SHARED((NP, H), jnp.float32),
            pltpu.SemaphoreType.DMA,
            pltpu.SemaphoreType.DMA,
        ],
    )
    def scatter_kernel(g_hbm, src2d_hbm, dst2d_hbm, zeros_hbm, part_hbm,
                       sidxb, didxb, rows, acc, sem0, sem1):
        cid = lax.axis_index("c")
        sid = lax.axis_index("s")
        wid = sid * NC + cid
        sems = [sem0, sem1]

        row0 = sid * RPT
        pltpu.sync_copy(zeros_hbm.at[pl.ds(row0, RPT)],
                        acc.at[pl.ds(row0, RPT)])
        plsc.subcore_barrier()

        def blk_body(t, _):
            blk = wid + NW * t
            pltpu.sync_copy(src2d_hbm.at[pl.ds(blk * BR, BR)], sidxb)
            pltpu.sync_copy(dst2d_hbm.at[pl.ds(blk * BR, BR)], didxb)

            def gather(j):
                return pltpu.async_copy(
                    g_hbm.at[sidxb.at[j]],
                    rows.at[j % 2], sems[j % 2])

            d = {0: gather(0)}
            for j in range(BR):
                if j + 1 < BR:
                    d[j + 1] = gather(j + 1)
                d[j].wait()
                pltpu.sync_copy(rows.at[j % 2], acc.at[didxb.at[j]],
                                add=True)
            return 0

        lax.fori_loop(0, T, blk_body, 0)
        plsc.subcore_barrier()
        pltpu.sync_copy(acc.at[pl.ds(row0, RPT)],
                        part_hbm.at[cid, pl.ds(row0, RPT)])

    return scatter_kernel


def _t1_body(x_ref, w_ref, degp_ref, g_ref, dinv_ref):
    n = x_ref.shape[0]
    deg = degp_ref[0, :n, 0:1] + degp_ref[1, :n, 0:1] + 1.0
    dinv = lax.rsqrt(deg)
    h = jnp.dot(x_ref[...], w_ref[...], preferred_element_type=jnp.float32)
    g_ref[...] = h * dinv
    dinv_ref[...] = dinv


def _t2_body(p_ref, g_ref, dinv_ref, b_ref, w_ref, g2_ref):
    n = g_ref.shape[0]
    dinv = dinv_ref[...]
    s = p_ref[0, :n] + p_ref[1, :n] + g_ref[...]
    z = jnp.maximum(dinv * s + b_ref[...], 0.0)
    g2_ref[...] = jnp.dot(z, w_ref[...],
                          preferred_element_type=jnp.float32) * dinv


def _t3_body(p_ref, g_ref, dinv_ref, b_ref, wc_ref, bc_ref, out_ref):
    n = g_ref.shape[0]
    dinv = dinv_ref[...]
    z = jnp.maximum(
        dinv * (p_ref[0, :n] + p_ref[1, :n] + g_ref[...]) + b_ref[...], 0.0)
    out_ref[...] = jnp.dot(z, wc_ref[...],
                           preferred_element_type=jnp.float32) + bc_ref[...]


def kernel(x, edge_index, W1, b1, W2, b2, Wc, bc):
    N, D = x.shape
    E = edge_index.shape[1]
    H = W1.shape[1]
    C = Wc.shape[1]
    f32 = jnp.float32

    NP = ((N + NS * 8 - 1) // (NS * 8)) * (NS * 8)
    src2d = edge_index[0].reshape(E // ROW, ROW)
    dst2d = edge_index[1].reshape(E // ROW, ROW)
    zeros = jnp.zeros((NP, H), f32)
    zeros1 = jnp.zeros((NP, DW), f32)
    ones_rows = jnp.ones((ROW, DW), f32)

    deg_kernel = _make_deg_kernel(NP, E)
    scatter_kernel = _make_scatter_kernel(NP, E, H)

    degp = deg_kernel(dst2d, ones_rows, zeros1)

    g1, dinv = pl.pallas_call(
        _t1_body,
        out_shape=[jax.ShapeDtypeStruct((N, H), f32),
                   jax.ShapeDtypeStruct((N, 1), f32)],
    )(x, W1, degp)

    p1 = scatter_kernel(g1, src2d, dst2d, zeros)

    g2 = pl.pallas_call(
        _t2_body,
        out_shape=jax.ShapeDtypeStruct((N, H), f32),
    )(p1, g1, dinv, b1, W2)

    p2 = scatter_kernel(g2, src2d, dst2d, zeros)

    logits = pl.pallas_call(
        _t3_body,
        out_shape=jax.ShapeDtypeStruct((N, C), f32),
    )(p2, g2, dinv, b2, Wc, bc)

    return logits


# speedup vs baseline: 28.4647x; 1.0329x over previous
import functools

import jax
import jax.numpy as jnp
from jax import lax
from jax.experimental import pallas as pl
from jax.experimental.pallas import tpu as pltpu
from jax.experimental.pallas import tpu_sc as plsc

NC = 2
NS = 16
NW = NC * NS
LANES = 16

ROW = 125
DW = 16


def _mesh():
    return plsc.VectorSubcoreMesh(core_axis_name="c", subcore_axis_name="s")


def _make_deg_kernel(NP, E):
    n_rows = E // ROW
    BR = 8
    n_blocks = n_rows // BR
    T = n_blocks // NW
    RPT = NP // NS
    assert E % ROW == 0 and n_rows % BR == 0 and n_blocks % NW == 0
    assert RPT % 8 == 0

    @functools.partial(
        pl.kernel,
        out_type=jax.ShapeDtypeStruct((NC, NP, DW), jnp.float32),
        mesh=_mesh(),
        compiler_params=pltpu.CompilerParams(use_tc_tiling_on_sc=False),
        scratch_types=[
            pltpu.VMEM((BR, ROW), jnp.int32),
            pltpu.VMEM((ROW, DW), jnp.float32),
            pltpu.VMEM_SHARED((NP, DW), jnp.float32),
        ],
    )
    def deg_kernel(dst2d_hbm, ones_hbm, zeros1_hbm, out_hbm,
                   didxb, ones_v, acc):
        cid = lax.axis_index("c")
        sid = lax.axis_index("s")
        wid = sid * NC + cid

        row0 = sid * RPT
        pltpu.sync_copy(zeros1_hbm.at[pl.ds(row0, RPT)],
                        acc.at[pl.ds(row0, RPT)])
        pltpu.sync_copy(ones_hbm, ones_v)
        plsc.subcore_barrier()

        def blk_body(t, _):
            blk = wid + NW * t
            pltpu.sync_copy(dst2d_hbm.at[pl.ds(blk * BR, BR)], didxb)
            for j in range(BR):
                pltpu.sync_copy(ones_v, acc.at[didxb.at[j]], add=True)
            return 0

        lax.fori_loop(0, T, blk_body, 0)
        plsc.subcore_barrier()
        pltpu.sync_copy(acc.at[pl.ds(row0, RPT)],
                        out_hbm.at[cid, pl.ds(row0, RPT)])

    return deg_kernel


def _make_scatter_kernel(NP, E, H):
    n_rows = E // ROW
    BR = 8
    n_blocks = n_rows // BR
    T = n_blocks // NW
    RPT = NP // NS
    assert E % ROW == 0 and n_rows % BR == 0 and n_blocks % NW == 0
    assert RPT % 8 == 0

    assert T % 2 == 0

    @functools.partial(
        pl.kernel,
        out_type=jax.ShapeDtypeStruct((NC, NP, H), jnp.float32),
        mesh=_mesh(),
        scratch_types=[
            pltpu.VMEM((2, BR, ROW), jnp.int32),
            pltpu.VMEM((2, BR, ROW), jnp.int32),
            pltpu.VMEM((2, ROW, H), jnp.float32),
            pltpu.VMEM_SHARED((NP, H), jnp.float32),
            pltpu.SemaphoreType.DMA,
            pltpu.SemaphoreType.DMA,
            pltpu.SemaphoreType.DMA,
            pltpu.SemaphoreType.DMA,
        ],
    )
    def scatter_kernel(g_hbm, src2d_hbm, dst2d_hbm, zeros_hbm, part_hbm,
                       sidxb, didxb, rows, acc, sg0, sg1, ss0, ss1):
        cid = lax.axis_index("c")
        sid = lax.axis_index("s")
        wid = sid * NC + cid
        sgs = [sg0, sg1]
        sss = [ss0, ss1]

        row0 = sid * RPT
        pltpu.sync_copy(zeros_hbm.at[pl.ds(row0, RPT)],
                        acc.at[pl.ds(row0, RPT)])
        plsc.subcore_barrier()

        def scat_wait(b, pb):
            pltpu.make_async_copy(rows.at[b], acc.at[didxb.at[pb, 0]],
                                  sss[b]).wait()

        def blk2_body(tt, _):
            for h in range(2):
                t = tt * 2 + h
                blk = wid + NW * t
                pltpu.sync_copy(src2d_hbm.at[pl.ds(blk * BR, BR)],
                                sidxb.at[h])
                pltpu.sync_copy(dst2d_hbm.at[pl.ds(blk * BR, BR)],
                                didxb.at[h])
                g = {}
                s = {}

                def issue_scatter(k, h=h):
                    return pltpu.async_copy(rows.at[k % 2],
                                            acc.at[didxb.at[h, k]],
                                            sss[k % 2], add=True)

                for j in range(BR):
                    b = j % 2
                    if j >= 2:
                        s[j - 2].wait()
                    elif h == 0:
                        @pl.when(tt > 0)
                        def _(b=b):
                            scat_wait(b, 1)
                    else:
                        scat_wait(b, 0)
                    g[j] = pltpu.async_copy(g_hbm.at[sidxb.at[h, j]],
                                            rows.at[b], sgs[b])
                    if j >= 1:
                        g[j - 1].wait()
                        s[j - 1] = issue_scatter(j - 1)
                g[BR - 1].wait()
                s[BR - 1] = issue_scatter(BR - 1)
            return 0

        lax.fori_loop(0, T // 2, blk2_body, 0)
        for b in range(2):
            scat_wait(b, 1)
        plsc.subcore_barrier()
        pltpu.sync_copy(acc.at[pl.ds(row0, RPT)],
                        part_hbm.at[cid, pl.ds(row0, RPT)])

    return scatter_kernel


def _t1_body(x_ref, w_ref, degp_ref, g_ref, dinv_ref):
    n = x_ref.shape[0]
    deg = degp_ref[0, :n, 0:1] + degp_ref[1, :n, 0:1] + 1.0
    dinv = lax.rsqrt(deg)
    h = jnp.dot(x_ref[...], w_ref[...], preferred_element_type=jnp.float32)
    g_ref[...] = h * dinv
    dinv_ref[...] = dinv


def _t2_body(p_ref, g_ref, dinv_ref, b_ref, w_ref, g2_ref):
    n = g_ref.shape[0]
    dinv = dinv_ref[...]
    s = p_ref[0, :n] + p_ref[1, :n] + g_ref[...]
    z = jnp.maximum(dinv * s + b_ref[...], 0.0)
    g2_ref[...] = jnp.dot(z, w_ref[...],
                          preferred_element_type=jnp.float32) * dinv


def _t3_body(p_ref, g_ref, dinv_ref, b_ref, wc_ref, bc_ref, out_ref):
    n = g_ref.shape[0]
    dinv = dinv_ref[...]
    z = jnp.maximum(
        dinv * (p_ref[0, :n] + p_ref[1, :n] + g_ref[...]) + b_ref[...], 0.0)
    out_ref[...] = jnp.dot(z, wc_ref[...],
                           preferred_element_type=jnp.float32) + bc_ref[...]


def kernel(x, edge_index, W1, b1, W2, b2, Wc, bc):
    N, D = x.shape
    E = edge_index.shape[1]
    H = W1.shape[1]
    C = Wc.shape[1]
    f32 = jnp.float32

    NP = ((N + NS * 8 - 1) // (NS * 8)) * (NS * 8)
    src2d = edge_index[0].reshape(E // ROW, ROW)
    dst2d = edge_index[1].reshape(E // ROW, ROW)
    zeros = jnp.zeros((NP, H), f32)
    zeros1 = jnp.zeros((NP, DW), f32)
    ones_rows = jnp.ones((ROW, DW), f32)

    deg_kernel = _make_deg_kernel(NP, E)
    scatter_kernel = _make_scatter_kernel(NP, E, H)

    degp = deg_kernel(dst2d, ones_rows, zeros1)

    g1, dinv = pl.pallas_call(
        _t1_body,
        out_shape=[jax.ShapeDtypeStruct((N, H), f32),
                   jax.ShapeDtypeStruct((N, 1), f32)],
    )(x, W1, degp)

    p1 = scatter_kernel(g1, src2d, dst2d, zeros)

    g2 = pl.pallas_call(
        _t2_body,
        out_shape=jax.ShapeDtypeStruct((N, H), f32),
    )(p1, g1, dinv, b1, W2)

    p2 = scatter_kernel(g2, src2d, dst2d, zeros)

    logits = pl.pallas_call(
        _t3_body,
        out_shape=jax.ShapeDtypeStruct((N, C), f32),
    )(p2, g2, dinv, b2, Wc, bc)

    return logits


# speedup vs baseline: 28.9850x; 1.0183x over previous
import functools

import jax
import jax.numpy as jnp
from jax import lax
from jax.experimental import pallas as pl
from jax.experimental.pallas import tpu as pltpu
from jax.experimental.pallas import tpu_sc as plsc

NC = 2
NS = 16
NW = NC * NS
LANES = 16

ROW = 125
DW = 16


def _mesh():
    return plsc.VectorSubcoreMesh(core_axis_name="c", subcore_axis_name="s")


def _make_deg_kernel(NP, E):
    n_rows = E // ROW
    BR = 8
    n_blocks = n_rows // BR
    T = n_blocks // NW
    RPT = NP // NS
    assert E % ROW == 0 and n_rows % BR == 0 and n_blocks % NW == 0
    assert RPT % 8 == 0

    @functools.partial(
        pl.kernel,
        out_type=jax.ShapeDtypeStruct((NC, NP, DW), jnp.float32),
        mesh=_mesh(),
        compiler_params=pltpu.CompilerParams(use_tc_tiling_on_sc=False),
        scratch_types=[
            pltpu.VMEM((2, BR, ROW), jnp.int32),
            pltpu.VMEM((ROW, DW), jnp.float32),
            pltpu.VMEM_SHARED((NP, DW), jnp.float32),
            pltpu.SemaphoreType.DMA,
            pltpu.SemaphoreType.DMA,
        ],
    )
    def deg_kernel(dst2d_hbm, ones_hbm, zeros1_hbm, out_hbm,
                   didxb, ones_v, acc, sd0, sd1):
        cid = lax.axis_index("c")
        sid = lax.axis_index("s")
        wid = sid * NC + cid
        sds = [sd0, sd1]
        assert T % 2 == 0

        row0 = sid * RPT
        pltpu.sync_copy(zeros1_hbm.at[pl.ds(row0, RPT)],
                        acc.at[pl.ds(row0, RPT)])
        pltpu.sync_copy(ones_hbm, ones_v)
        plsc.subcore_barrier()

        def drain(h):
            for _ in range(BR):
                pltpu.make_async_copy(ones_v, acc.at[didxb.at[h, 0]],
                                      sds[h]).wait()

        def blk2_body(tt, _):
            for h in range(2):
                t = tt * 2 + h

                @pl.when(tt > 0)
                def _(h=h):
                    drain(h)

                blk = wid + NW * t
                pltpu.sync_copy(dst2d_hbm.at[pl.ds(blk * BR, BR)],
                                didxb.at[h])
                for j in range(BR):
                    pltpu.async_copy(ones_v, acc.at[didxb.at[h, j]],
                                     sds[h], add=True)
            return 0

        lax.fori_loop(0, T // 2, blk2_body, 0)
        drain(0)
        drain(1)
        plsc.subcore_barrier()
        pltpu.sync_copy(acc.at[pl.ds(row0, RPT)],
                        out_hbm.at[cid, pl.ds(row0, RPT)])

    return deg_kernel


def _make_scatter_kernel(NP, E, H):
    n_rows = E // ROW
    BR = 8
    n_blocks = n_rows // BR
    T = n_blocks // NW
    RPT = NP // NS
    assert E % ROW == 0 and n_rows % BR == 0 and n_blocks % NW == 0
    assert RPT % 8 == 0

    assert T % 2 == 0

    @functools.partial(
        pl.kernel,
        out_type=jax.ShapeDtypeStruct((NC, NP, H), jnp.float32),
        mesh=_mesh(),
        scratch_types=[
            pltpu.VMEM((2, BR, ROW), jnp.int32),
            pltpu.VMEM((2, BR, ROW), jnp.int32),
            pltpu.VMEM((2, ROW, H), jnp.float32),
            pltpu.VMEM_SHARED((NP, H), jnp.float32),
            pltpu.SemaphoreType.DMA,
            pltpu.SemaphoreType.DMA,
            pltpu.SemaphoreType.DMA,
            pltpu.SemaphoreType.DMA,
        ],
    )
    def scatter_kernel(g_hbm, src2d_hbm, dst2d_hbm, zeros_hbm, part_hbm,
                       sidxb, didxb, rows, acc, sg0, sg1, ss0, ss1):
        cid = lax.axis_index("c")
        sid = lax.axis_index("s")
        wid = sid * NC + cid
        sgs = [sg0, sg1]
        sss = [ss0, ss1]

        row0 = sid * RPT
        pltpu.sync_copy(zeros_hbm.at[pl.ds(row0, RPT)],
                        acc.at[pl.ds(row0, RPT)])
        plsc.subcore_barrier()

        def scat_wait(b, pb):
            pltpu.make_async_copy(rows.at[b], acc.at[didxb.at[pb, 0]],
                                  sss[b]).wait()

        def blk2_body(tt, _):
            for h in range(2):
                t = tt * 2 + h
                blk = wid + NW * t
                pltpu.sync_copy(src2d_hbm.at[pl.ds(blk * BR, BR)],
                                sidxb.at[h])
                pltpu.sync_copy(dst2d_hbm.at[pl.ds(blk * BR, BR)],
                                didxb.at[h])
                g = {}
                s = {}

                def issue_scatter(k, h=h):
                    return pltpu.async_copy(rows.at[k % 2],
                                            acc.at[didxb.at[h, k]],
                                            sss[k % 2], add=True)

                for j in range(BR):
                    b = j % 2
                    if j >= 2:
                        s[j - 2].wait()
                    elif h == 0:
                        @pl.when(tt > 0)
                        def _(b=b):
                            scat_wait(b, 1)
                    else:
                        scat_wait(b, 0)
                    g[j] = pltpu.async_copy(g_hbm.at[sidxb.at[h, j]],
                                            rows.at[b], sgs[b])
                    if j >= 1:
                        g[j - 1].wait()
                        s[j - 1] = issue_scatter(j - 1)
                g[BR - 1].wait()
                s[BR - 1] = issue_scatter(BR - 1)
            return 0

        lax.fori_loop(0, T // 2, blk2_body, 0)
        for b in range(2):
            scat_wait(b, 1)
        plsc.subcore_barrier()
        pltpu.sync_copy(acc.at[pl.ds(row0, RPT)],
                        part_hbm.at[cid, pl.ds(row0, RPT)])

    return scatter_kernel


def _t1a_body(x_ref, w_ref, h_ref):
    h_ref[...] = jnp.dot(x_ref[...], w_ref[...],
                         preferred_element_type=jnp.float32)


def _t1b_body(h_ref, degp_ref, g_ref, dinv_ref):
    n = h_ref.shape[0]
    deg = degp_ref[0, :n, 0:1] + degp_ref[1, :n, 0:1] + 1.0
    dinv = lax.rsqrt(deg)
    g_ref[...] = h_ref[...] * dinv
    dinv_ref[...] = dinv


def _t2_body(p_ref, g_ref, dinv_ref, b_ref, w_ref, g2_ref):
    n = g_ref.shape[0]
    dinv = dinv_ref[...]
    s = p_ref[0, :n] + p_ref[1, :n] + g_ref[...]
    z = jnp.maximum(dinv * s + b_ref[...], 0.0)
    g2_ref[...] = jnp.dot(z, w_ref[...],
                          preferred_element_type=jnp.float32) * dinv


def _t3_body(p_ref, g_ref, dinv_ref, b_ref, wc_ref, bc_ref, out_ref):
    n = g_ref.shape[0]
    dinv = dinv_ref[...]
    z = jnp.maximum(
        dinv * (p_ref[0, :n] + p_ref[1, :n] + g_ref[...]) + b_ref[...], 0.0)
    out_ref[...] = jnp.dot(z, wc_ref[...],
                           preferred_element_type=jnp.float32) + bc_ref[...]


def kernel(x, edge_index, W1, b1, W2, b2, Wc, bc):
    N, D = x.shape
    E = edge_index.shape[1]
    H = W1.shape[1]
    C = Wc.shape[1]
    f32 = jnp.float32

    NP = ((N + NS * 8 - 1) // (NS * 8)) * (NS * 8)
    src2d = edge_index[0].reshape(E // ROW, ROW)
    dst2d = edge_index[1].reshape(E // ROW, ROW)
    zeros = jnp.zeros((NP, H), f32)
    zeros1 = jnp.zeros((NP, DW), f32)
    ones_rows = jnp.ones((ROW, DW), f32)

    deg_kernel = _make_deg_kernel(NP, E)
    scatter_kernel = _make_scatter_kernel(NP, E, H)

    degp = deg_kernel(dst2d, ones_rows, zeros1)

    h1 = pl.pallas_call(
        _t1a_body,
        out_shape=jax.ShapeDtypeStruct((N, H), f32),
    )(x, W1)

    g1, dinv = pl.pallas_call(
        _t1b_body,
        out_shape=[jax.ShapeDtypeStruct((N, H), f32),
                   jax.ShapeDtypeStruct((N, 1), f32)],
    )(h1, degp)

    p1 = scatter_kernel(g1, src2d, dst2d, zeros)

    g2 = pl.pallas_call(
        _t2_body,
        out_shape=jax.ShapeDtypeStruct((N, H), f32),
    )(p1, g1, dinv, b1, W2)

    p2 = scatter_kernel(g2, src2d, dst2d, zeros)

    logits = pl.pallas_call(
        _t3_body,
        out_shape=jax.ShapeDtypeStruct((N, C), f32),
    )(p2, g2, dinv, b2, Wc, bc)

    return logits
